# baseline (speedup 1.0000x reference)
import functools

import jax
import jax.numpy as jnp
from jax import lax
from jax.experimental import pallas as pl
from jax.experimental.pallas import tpu as pltpu
from jax.experimental.pallas import tpu_sc as plsc

_SC_CORES = 2
_SC_SUBCORES = 16
_SC_WORKERS = _SC_CORES * _SC_SUBCORES


def _gate(x_flat, Wg, bg, *, tb, interpret=False):
    n, d = x_flat.shape
    e = Wg.shape[1]
    gb = n // tb

    def body(x_ref, wg_ref, bg_ref, eid_ref, score_ref, rank_ref, counts_ref,
             tril_ref):
        i = pl.program_id(0)
        s = jnp.dot(x_ref[...], wg_ref[...], preferred_element_type=jnp.float32)
        s = s + bg_ref[...]
        m = jnp.max(s, axis=1, keepdims=True)
        lane = lax.broadcasted_iota(jnp.int32, (tb, e), 1)
        eid = jnp.min(jnp.where(s >= m, lane, e), axis=1)
        oneh = (lane == eid[:, None]).astype(jnp.float32)

        @pl.when(i == 0)
        def _():
            counts_ref[...] = jnp.zeros_like(counts_ref)
            r_i = lax.broadcasted_iota(jnp.int32, (tb, tb), 0)
            c_i = lax.broadcasted_iota(jnp.int32, (tb, tb), 1)
            tril_ref[...] = (c_i < r_i).astype(jnp.float32)

        excl = jnp.dot(tril_ref[...], oneh, preferred_element_type=jnp.float32)

        rank = jnp.sum(oneh * (excl + counts_ref[...]), axis=1)
        counts_ref[...] = counts_ref[...] + jnp.sum(oneh, axis=0, keepdims=True)
        eid_ref[0, :, :] = eid[None, :]
        score_ref[0, :, :] = m[:, 0][None, :]
        rank_ref[0, :, :] = rank.astype(jnp.int32)[None, :]

    eid, score, rank, counts = pl.pallas_call(
        body,
        grid=(gb,),
        in_specs=[
            pl.BlockSpec((tb, d), lambda i: (i, 0)),
            pl.BlockSpec((d, e), lambda i: (0, 0)),
            pl.BlockSpec((1, e), lambda i: (0, 0)),
        ],
        out_specs=[
            pl.BlockSpec((1, 1, tb), lambda i: (i, 0, 0)),
            pl.BlockSpec((1, 1, tb), lambda i: (i, 0, 0)),
            pl.BlockSpec((1, 1, tb), lambda i: (i, 0, 0)),
            pl.BlockSpec((1, e), lambda i: (0, 0)),
        ],
        out_shape=[
            jax.ShapeDtypeStruct((gb, 1, tb), jnp.int32),
            jax.ShapeDtypeStruct((gb, 1, tb), jnp.float32),
            jax.ShapeDtypeStruct((gb, 1, tb), jnp.int32),
            jax.ShapeDtypeStruct((1, e), jnp.float32),
        ],
        scratch_shapes=[pltpu.VMEM((tb, tb), jnp.float32)],
        interpret=interpret,
    )(x_flat, Wg, bg.reshape(1, e))
    return eid, score, rank, counts


def _metadata(eid, score, rank, counts, *, bt, max_tiles, tb, sw,
              interpret=False):
    e = counts.shape[1]
    gb = eid.shape[0]
    n = gb * tb
    mt = max_tiles

    def body(counts_ref, eid_ref, rank_ref, score_ref, pos_ref, s16_ref,
             te_ref, tv_ref, tm_ref):
        i = pl.program_id(0)
        cnt = counts_ref[...]
        tiles_e = jnp.floor((cnt + (bt - 1)) * (1.0 / bt))
        r_i = lax.broadcasted_iota(jnp.int32, (e, e), 0)
        c_i = lax.broadcasted_iota(jnp.int32, (e, e), 1)
        ut = (r_i <= c_i).astype(jnp.float32)
        tile_cum = jnp.dot(tiles_e, ut,
                           preferred_element_type=jnp.float32)
        row_start = (tile_cum - tiles_e) * bt
        total = tile_cum[:, e - 1:e]

        eid_b = eid_ref[0, 0, :]
        lane = lax.broadcasted_iota(jnp.int32, (tb, e), 1)
        oneh = (lane == eid_b[:, None]).astype(jnp.float32)
        posf = jnp.sum(oneh * row_start, axis=1)
        pos_ref[0, :, :] = (posf.astype(jnp.int32) + rank_ref[0, :, :])
        s16_ref[...] = jnp.broadcast_to(score_ref[0, 0, :][:, None], (tb, sw))

        @pl.when(i == 0)
        def _():
            tj = lax.broadcasted_iota(jnp.int32, (mt, 1), 0).astype(
                jnp.float32)
            cmp = (jnp.broadcast_to(tile_cum, (mt, e)) <= tj)
            e_all = jnp.sum(cmp.astype(jnp.float32), axis=1,
                            keepdims=True)
            valid = tj < total
            last = jnp.maximum(total - 1.0, 0.0)
            te_ref[...] = jnp.where(valid, e_all, last).astype(jnp.int32)[:, 0][None, :]
            tv_ref[...] = valid.astype(jnp.int32)[:, 0][None, :]
            tm_ref[...] = jnp.where(valid, tj, last).astype(jnp.int32)[:, 0][None, :]

    pos, s16, te, tv, tm = pl.pallas_call(
        body,
        grid=(gb,),
        in_specs=[
            pl.BlockSpec((1, e), lambda i: (0, 0)),
            pl.BlockSpec((1, 1, tb), lambda i: (i, 0, 0)),
            pl.BlockSpec((1, 1, tb), lambda i: (i, 0, 0)),
            pl.BlockSpec((1, 1, tb), lambda i: (i, 0, 0)),
        ],
        out_specs=[
            pl.BlockSpec((1, 1, tb), lambda i: (i, 0, 0)),
            pl.BlockSpec((tb, sw), lambda i: (i, 0)),
            pl.BlockSpec((1, mt), lambda i: (0, 0)),
            pl.BlockSpec((1, mt), lambda i: (0, 0)),
            pl.BlockSpec((1, mt), lambda i: (0, 0)),
        ],
        out_shape=[
            jax.ShapeDtypeStruct((gb, 1, tb), jnp.int32),
            jax.ShapeDtypeStruct((n, sw), jnp.float32),
            jax.ShapeDtypeStruct((1, mt), jnp.int32),
            jax.ShapeDtypeStruct((1, mt), jnp.int32),
            jax.ShapeDtypeStruct((1, mt), jnp.int32),
        ],
        interpret=interpret,
    )(counts.reshape(1, e), eid, rank, score)
    return (pos.reshape(n), s16, te.reshape(mt), tv.reshape(mt),
            tm.reshape(mt))


def _sc_gather(table, idx, n_used=None, *, chunk=64):
    r, d = table.shape
    n = idx.shape[0]
    per = n // _SC_WORKERS
    c = min(chunk, per)
    steps = per // c
    if n_used is None:
        n_used = jnp.full((16,), n, jnp.int32)
    mesh = plsc.VectorSubcoreMesh(core_axis_name="c", subcore_axis_name="s")

    @functools.partial(
        pl.kernel,
        mesh=mesh,
        out_type=jax.ShapeDtypeStruct((n, d), jnp.float32),
        scratch_types=[
            pltpu.VMEM((c,), jnp.int32),
            pltpu.VMEM((c, d), jnp.float32),
            pltpu.VMEM((16,), jnp.int32),
            pltpu.SemaphoreType.DMA,
        ],
    )
    def gather_k(table_hbm, idx_hbm, nu_hbm, out_hbm, idx_v, rows_v, nu_v,
                 sem):
        wid = lax.axis_index("s") * _SC_CORES + lax.axis_index("c")
        base = wid * per
        pltpu.sync_copy(nu_hbm, nu_v)
        rem = nu_v[...][0] - base
        my_steps = jnp.clip((rem + c - 1) // c, 0, steps)

        def body(k, carry):
            b = base + k * c
            pltpu.sync_copy(idx_hbm.at[pl.ds(b, c)], idx_v)
            pltpu.async_copy(table_hbm.at[idx_v], rows_v, sem).wait()
            pltpu.sync_copy(rows_v, out_hbm.at[pl.ds(b, c)])
            return carry

        lax.fori_loop(0, my_steps, body, 0)

    return gather_k(table, idx, n_used)


def _sc_scatter_rows(x, s16, pos, n_pad, *, chunk=64):
    n, d = x.shape
    sw = s16.shape[1]
    per = n // _SC_WORKERS
    c = min(chunk, per)
    steps = per // c
    mesh = plsc.VectorSubcoreMesh(core_axis_name="c", subcore_axis_name="s")

    @functools.partial(
        pl.kernel,
        mesh=mesh,
        out_type=[
            jax.ShapeDtypeStruct((n_pad, d), x.dtype),
            jax.ShapeDtypeStruct((n_pad, sw), jnp.float32),
        ],
        scratch_types=[
            pltpu.VMEM((c,), jnp.int32),
            pltpu.VMEM((c, d), x.dtype),
            pltpu.VMEM((c, sw), jnp.float32),
            pltpu.SemaphoreType.DMA,
        ],
    )
    def scatter_k(x_hbm, s_hbm, pos_hbm, xs_hbm, ss_hbm, idx_v, rows_v, s_v,
                  sem):
        wid = lax.axis_index("s") * _SC_CORES + lax.axis_index("c")
        base = wid * per

        def body(k, carry):
            b = base + k * c
            pltpu.sync_copy(pos_hbm.at[pl.ds(b, c)], idx_v)
            pltpu.sync_copy(x_hbm.at[pl.ds(b, c)], rows_v)
            pltpu.sync_copy(s_hbm.at[pl.ds(b, c)], s_v)
            pltpu.async_copy(rows_v, xs_hbm.at[idx_v], sem).wait()
            pltpu.async_copy(s_v, ss_hbm.at[idx_v], sem).wait()
            return carry

        lax.fori_loop(0, steps, body, 0)

    return scatter_k(x, s16, pos)


def _experts(xs, W1, b1, W2, b2, score_col, tile_eid, tile_valid, tile_map,
             *, bt, interpret=False):
    e, d, h = W1.shape
    n_pad = xs.shape[0]
    max_tiles = n_pad // bt

    def body(te_ref, tv_ref, tm_ref, xs_ref, w1_ref, b1_ref, w2_ref, b2_ref,
             sc_ref, ys_ref):
        i = pl.program_id(0)

        @pl.when(tv_ref[i] == 1)
        def _():
            hact = jnp.dot(xs_ref[...], w1_ref[0],
                           preferred_element_type=jnp.float32) + b1_ref[0]
            hact = jnp.maximum(hact, 0.0)
            y = jnp.dot(hact, w2_ref[0],
                        preferred_element_type=jnp.float32) + b2_ref[0]
            ys_ref[...] = y * sc_ref[...][:, :1]

    grid_spec = pltpu.PrefetchScalarGridSpec(
        num_scalar_prefetch=3,
        grid=(max_tiles,),
        in_specs=[
            pl.BlockSpec((bt, d), lambda i, te, tv, tm: (tm[i], 0)),
            pl.BlockSpec((1, d, h), lambda i, te, tv, tm: (te[i], 0, 0)),
            pl.BlockSpec((1, 1, h), lambda i, te, tv, tm: (te[i], 0, 0)),
            pl.BlockSpec((1, h, d), lambda i, te, tv, tm: (te[i], 0, 0)),
            pl.BlockSpec((1, 1, d), lambda i, te, tv, tm: (te[i], 0, 0)),
            pl.BlockSpec((bt, 128), lambda i, te, tv, tm: (tm[i], 0)),
        ],
        out_specs=pl.BlockSpec((bt, d), lambda i, te, tv, tm: (tm[i], 0)),
    )
    return pl.pallas_call(
        body,
        grid_spec=grid_spec,
        out_shape=jax.ShapeDtypeStruct((n_pad, d), jnp.float32),
        interpret=interpret,
    )(tile_eid, tile_valid, tile_map, xs, W1, b1.reshape(e, 1, h), W2,
      b2.reshape(e, 1, d), score_col)


def kernel(x, W1, b1, W2, b2, Wg, bg):
    bsz, t, d = x.shape
    e, _, h = W1.shape
    n = bsz * t
    bt = 256
    max_tiles = n // bt + e
    x_flat = x.reshape(n, d)

    eid, score, rank, counts = _gate(x_flat, Wg, bg, tb=1024)
    pos, s16, tile_eid, tile_valid, tile_map = _metadata(
        eid, score, rank, counts, bt=bt, max_tiles=max_tiles, tb=1024, sw=128)
    xs, ss = _sc_scatter_rows(x_flat, s16, pos, max_tiles * bt)
    ys = _experts(xs, W1, b1, W2, b2, ss, tile_eid, tile_valid, tile_map,
                  bt=bt)
    out = _sc_gather(ys, pos)
    return out.reshape(bsz, t, d)


# speedup vs baseline: 1.0036x; 1.0036x over previous
import functools

import jax
import jax.numpy as jnp
from jax import lax
from jax.experimental import pallas as pl
from jax.experimental.pallas import tpu as pltpu
from jax.experimental.pallas import tpu_sc as plsc

_SC_CORES = 2
_SC_SUBCORES = 16
_SC_WORKERS = _SC_CORES * _SC_SUBCORES


def _gate(x_flat, Wg, bg, *, tb, interpret=False):
    n, d = x_flat.shape
    e = Wg.shape[1]
    gb = n // tb

    def body(x_ref, wg_ref, bg_ref, eid_ref, score_ref, rank_ref, counts_ref,
             tril_ref):
        i = pl.program_id(0)
        s = jnp.dot(x_ref[...], wg_ref[...], preferred_element_type=jnp.float32)
        s = s + bg_ref[...]
        m = jnp.max(s, axis=1, keepdims=True)
        lane = lax.broadcasted_iota(jnp.int32, (tb, e), 1)
        eid = jnp.min(jnp.where(s >= m, lane, e), axis=1)
        oneh = (lane == eid[:, None]).astype(jnp.float32)

        @pl.when(i == 0)
        def _():
            counts_ref[...] = jnp.zeros_like(counts_ref)
            r_i = lax.broadcasted_iota(jnp.int32, (tb, tb), 0)
            c_i = lax.broadcasted_iota(jnp.int32, (tb, tb), 1)
            tril_ref[...] = (c_i < r_i).astype(jnp.float32)

        excl = jnp.dot(tril_ref[...], oneh, preferred_element_type=jnp.float32)

        rank = jnp.sum(oneh * (excl + counts_ref[...]), axis=1)
        counts_ref[...] = counts_ref[...] + jnp.sum(oneh, axis=0, keepdims=True)
        eid_ref[0, :, :] = eid[None, :]
        score_ref[0, :, :] = m[:, 0][None, :]
        rank_ref[0, :, :] = rank.astype(jnp.int32)[None, :]

    eid, score, rank, counts = pl.pallas_call(
        body,
        grid=(gb,),
        in_specs=[
            pl.BlockSpec((tb, d), lambda i: (i, 0)),
            pl.BlockSpec((d, e), lambda i: (0, 0)),
            pl.BlockSpec((1, e), lambda i: (0, 0)),
        ],
        out_specs=[
            pl.BlockSpec((1, 1, tb), lambda i: (i, 0, 0)),
            pl.BlockSpec((1, 1, tb), lambda i: (i, 0, 0)),
            pl.BlockSpec((1, 1, tb), lambda i: (i, 0, 0)),
            pl.BlockSpec((1, e), lambda i: (0, 0)),
        ],
        out_shape=[
            jax.ShapeDtypeStruct((gb, 1, tb), jnp.int32),
            jax.ShapeDtypeStruct((gb, 1, tb), jnp.float32),
            jax.ShapeDtypeStruct((gb, 1, tb), jnp.int32),
            jax.ShapeDtypeStruct((1, e), jnp.float32),
        ],
        scratch_shapes=[pltpu.VMEM((tb, tb), jnp.float32)],
        interpret=interpret,
    )(x_flat, Wg, bg.reshape(1, e))
    return eid, score, rank, counts


def _metadata(eid, score, rank, counts, *, bt, max_tiles, tb, sw,
              interpret=False):
    e = counts.shape[1]
    gb = eid.shape[0]
    n = gb * tb
    mt = max_tiles

    def body(counts_ref, eid_ref, rank_ref, score_ref, pos_ref, s16_ref,
             te_ref, tv_ref, tm_ref):
        i = pl.program_id(0)
        cnt = counts_ref[...]
        tiles_e = jnp.floor((cnt + (bt - 1)) * (1.0 / bt))
        r_i = lax.broadcasted_iota(jnp.int32, (e, e), 0)
        c_i = lax.broadcasted_iota(jnp.int32, (e, e), 1)
        ut = (r_i <= c_i).astype(jnp.float32)
        tile_cum = jnp.dot(tiles_e, ut,
                           preferred_element_type=jnp.float32)
        row_start = (tile_cum - tiles_e) * bt
        total = tile_cum[:, e - 1:e]

        eid_b = eid_ref[0, 0, :]
        lane = lax.broadcasted_iota(jnp.int32, (tb, e), 1)
        oneh = (lane == eid_b[:, None]).astype(jnp.float32)
        posf = jnp.sum(oneh * row_start, axis=1)
        pos_ref[0, :, :] = (posf.astype(jnp.int32) + rank_ref[0, :, :])
        s16_ref[...] = jnp.broadcast_to(score_ref[0, 0, :][:, None], (tb, sw))

        @pl.when(i == 0)
        def _():
            tj = lax.broadcasted_iota(jnp.int32, (mt, 1), 0).astype(
                jnp.float32)
            cmp = (jnp.broadcast_to(tile_cum, (mt, e)) <= tj)
            e_all = jnp.sum(cmp.astype(jnp.float32), axis=1,
                            keepdims=True)
            valid = tj < total
            last = jnp.maximum(total - 1.0, 0.0)
            te_ref[...] = jnp.where(valid, e_all, last).astype(jnp.int32)[:, 0][None, :]
            tv_ref[...] = valid.astype(jnp.int32)[:, 0][None, :]
            tm_ref[...] = jnp.where(valid, tj, last).astype(jnp.int32)[:, 0][None, :]

    pos, s16, te, tv, tm = pl.pallas_call(
        body,
        grid=(gb,),
        in_specs=[
            pl.BlockSpec((1, e), lambda i: (0, 0)),
            pl.BlockSpec((1, 1, tb), lambda i: (i, 0, 0)),
            pl.BlockSpec((1, 1, tb), lambda i: (i, 0, 0)),
            pl.BlockSpec((1, 1, tb), lambda i: (i, 0, 0)),
        ],
        out_specs=[
            pl.BlockSpec((1, 1, tb), lambda i: (i, 0, 0)),
            pl.BlockSpec((tb, sw), lambda i: (i, 0)),
            pl.BlockSpec((1, mt), lambda i: (0, 0)),
            pl.BlockSpec((1, mt), lambda i: (0, 0)),
            pl.BlockSpec((1, mt), lambda i: (0, 0)),
        ],
        out_shape=[
            jax.ShapeDtypeStruct((gb, 1, tb), jnp.int32),
            jax.ShapeDtypeStruct((n, sw), jnp.float32),
            jax.ShapeDtypeStruct((1, mt), jnp.int32),
            jax.ShapeDtypeStruct((1, mt), jnp.int32),
            jax.ShapeDtypeStruct((1, mt), jnp.int32),
        ],
        interpret=interpret,
    )(counts.reshape(1, e), eid, rank, score)
    return (pos.reshape(n), s16, te.reshape(mt), tv.reshape(mt),
            tm.reshape(mt))


def _sc_gather(table, idx, *, chunk=32):
    r, d = table.shape
    n = idx.shape[0]
    per = n // _SC_WORKERS
    c = min(chunk, per)
    steps = per // c
    mesh = plsc.VectorSubcoreMesh(core_axis_name="c", subcore_axis_name="s")

    @functools.partial(
        pl.kernel,
        mesh=mesh,
        out_type=jax.ShapeDtypeStruct((n, d), jnp.float32),
        scratch_types=[
            pltpu.VMEM((c,), jnp.int32),
            pltpu.VMEM((c,), jnp.int32),
            pltpu.VMEM((c, d), jnp.float32),
            pltpu.VMEM((c, d), jnp.float32),
            pltpu.SemaphoreType.DMA,
            pltpu.SemaphoreType.DMA,
            pltpu.SemaphoreType.DMA,
            pltpu.SemaphoreType.DMA,
        ],
    )
    def gather_k(table_hbm, idx_hbm, out_hbm, idx0, idx1, rows0, rows1,
                 gs0, gs1, ss0, ss1):
        wid = lax.axis_index("s") * _SC_CORES + lax.axis_index("c")
        base = wid * per
        idx_v = [idx0, idx1]
        rows_v = [rows0, rows1]
        gsem = [gs0, gs1]
        ssem = [ss0, ss1]
        gh = [None] * steps
        sh = [None] * steps
        for k in range(steps):
            b = k & 1
            if k >= 2:
                sh[k - 2].wait()
            pltpu.sync_copy(idx_hbm.at[pl.ds(base + k * c, c)], idx_v[b])
            gh[k] = pltpu.async_copy(table_hbm.at[idx_v[b]], rows_v[b],
                                     gsem[b])
            if k >= 1:
                gh[k - 1].wait()
                sh[k - 1] = pltpu.async_copy(
                    rows_v[1 - b], out_hbm.at[pl.ds(base + (k - 1) * c, c)],
                    ssem[1 - b])
        lb = (steps - 1) & 1
        gh[steps - 1].wait()
        sh[steps - 1] = pltpu.async_copy(
            rows_v[lb], out_hbm.at[pl.ds(base + (steps - 1) * c, c)],
            ssem[lb])
        if steps >= 2:
            sh[steps - 2].wait()
        sh[steps - 1].wait()

    return gather_k(table, idx)


def _sc_scatter_rows(x, s16, pos, n_pad, *, chunk=32):
    n, d = x.shape
    sw = s16.shape[1]
    per = n // _SC_WORKERS
    c = min(chunk, per)
    steps = per // c
    mesh = plsc.VectorSubcoreMesh(core_axis_name="c", subcore_axis_name="s")

    @functools.partial(
        pl.kernel,
        mesh=mesh,
        out_type=[
            jax.ShapeDtypeStruct((n_pad, d), x.dtype),
            jax.ShapeDtypeStruct((n_pad, sw), jnp.float32),
        ],
        scratch_types=[
            pltpu.VMEM((c,), jnp.int32),
            pltpu.VMEM((c,), jnp.int32),
            pltpu.VMEM((c, d), x.dtype),
            pltpu.VMEM((c, d), x.dtype),
            pltpu.VMEM((c, sw), jnp.float32),
            pltpu.VMEM((c, sw), jnp.float32),
            pltpu.SemaphoreType.DMA,
            pltpu.SemaphoreType.DMA,
            pltpu.SemaphoreType.DMA,
            pltpu.SemaphoreType.DMA,
        ],
    )
    def scatter_k(x_hbm, s_hbm, pos_hbm, xs_hbm, ss_hbm, idx0, idx1, rows0,
                  rows1, s0, s1, xs0, xs1, sc0, sc1):
        wid = lax.axis_index("s") * _SC_CORES + lax.axis_index("c")
        base = wid * per
        idx_v = [idx0, idx1]
        rows_v = [rows0, rows1]
        s_v = [s0, s1]
        xsem = [xs0, xs1]
        ssem = [sc0, sc1]
        xh = [None] * steps
        sh = [None] * steps
        for k in range(steps):
            b = k & 1
            if k >= 2:
                xh[k - 2].wait()
                sh[k - 2].wait()
            off = base + k * c
            pltpu.sync_copy(pos_hbm.at[pl.ds(off, c)], idx_v[b])
            pltpu.sync_copy(x_hbm.at[pl.ds(off, c)], rows_v[b])
            pltpu.sync_copy(s_hbm.at[pl.ds(off, c)], s_v[b])
            xh[k] = pltpu.async_copy(rows_v[b], xs_hbm.at[idx_v[b]], xsem[b])
            sh[k] = pltpu.async_copy(s_v[b], ss_hbm.at[idx_v[b]], ssem[b])
        for k in range(max(steps - 2, 0), steps):
            xh[k].wait()
            sh[k].wait()

    return scatter_k(x, s16, pos)


def _experts(xs, W1, b1, W2, b2, score_col, tile_eid, tile_valid, tile_map,
             *, bt, interpret=False):
    e, d, h = W1.shape
    n_pad = xs.shape[0]
    max_tiles = n_pad // bt

    def body(te_ref, tv_ref, tm_ref, xs_ref, w1_ref, b1_ref, w2_ref, b2_ref,
             sc_ref, ys_ref):
        i = pl.program_id(0)

        @pl.when(tv_ref[i] == 1)
        def _():
            hact = jnp.dot(xs_ref[...], w1_ref[0],
                           preferred_element_type=jnp.float32) + b1_ref[0]
            hact = jnp.maximum(hact, 0.0)
            y = jnp.dot(hact, w2_ref[0],
                        preferred_element_type=jnp.float32) + b2_ref[0]
            ys_ref[...] = y * sc_ref[...][:, :1]

    grid_spec = pltpu.PrefetchScalarGridSpec(
        num_scalar_prefetch=3,
        grid=(max_tiles,),
        in_specs=[
            pl.BlockSpec((bt, d), lambda i, te, tv, tm: (tm[i], 0)),
            pl.BlockSpec((1, d, h), lambda i, te, tv, tm: (te[i], 0, 0)),
            pl.BlockSpec((1, 1, h), lambda i, te, tv, tm: (te[i], 0, 0)),
            pl.BlockSpec((1, h, d), lambda i, te, tv, tm: (te[i], 0, 0)),
            pl.BlockSpec((1, 1, d), lambda i, te, tv, tm: (te[i], 0, 0)),
            pl.BlockSpec((bt, 128), lambda i, te, tv, tm: (tm[i], 0)),
        ],
        out_specs=pl.BlockSpec((bt, d), lambda i, te, tv, tm: (tm[i], 0)),
    )
    return pl.pallas_call(
        body,
        grid_spec=grid_spec,
        out_shape=jax.ShapeDtypeStruct((n_pad, d), jnp.float32),
        interpret=interpret,
    )(tile_eid, tile_valid, tile_map, xs, W1, b1.reshape(e, 1, h), W2,
      b2.reshape(e, 1, d), score_col)


def kernel(x, W1, b1, W2, b2, Wg, bg):
    bsz, t, d = x.shape
    e, _, h = W1.shape
    n = bsz * t
    bt = 256
    max_tiles = n // bt + e
    x_flat = x.reshape(n, d)

    eid, score, rank, counts = _gate(x_flat, Wg, bg, tb=1024)
    pos, s16, tile_eid, tile_valid, tile_map = _metadata(
        eid, score, rank, counts, bt=bt, max_tiles=max_tiles, tb=1024, sw=128)
    xs, ss = _sc_scatter_rows(x_flat, s16, pos, max_tiles * bt)
    ys = _experts(xs, W1, b1, W2, b2, ss, tile_eid, tile_valid, tile_map,
                  bt=bt)
    out = _sc_gather(ys, pos)
    return out.reshape(bsz, t, d)
